# baseline (speedup 1.0000x reference)
import functools

import jax
import jax.numpy as jnp
from jax import lax
from jax.experimental import pallas as pl
from jax.experimental.pallas import tpu as pltpu
from jax.experimental.pallas import tpu_sc as plsc

N = 10000
EPS = 1e-5
LANES = 128
BE = 64
SCH = 32
NB = 4
LAG = 3
BED = 128
N_PAD = 10240
N_STRIPE = N_PAD // 16
NC, NS = 2, 16


def _sc_mesh():
    return plsc.VectorSubcoreMesh(
        core_axis_name="c", subcore_axis_name="s", num_cores=NC, num_subcores=NS)


def _deg_count(dst2, ones128, zeros128):
    kpt = dst2.shape[0] // (NC * NS)

    @functools.partial(
        pl.kernel,
        out_type=jax.ShapeDtypeStruct((NC, N_PAD, LANES), jnp.float32),
        mesh=_sc_mesh(),
        scratch_types=[
            pltpu.VMEM((kpt, BED), jnp.int32),
            pltpu.VMEM((BED, LANES), jnp.float32),
            pltpu.VMEM_SHARED((N_PAD, LANES), jnp.float32),
            pltpu.SemaphoreType.DMA,
        ],
    )
    def deg_kernel(dst_hbm, ones_hbm, zeros_hbm, out_hbm, idx_v, ones_v, acc, sem):
        c = lax.axis_index("c")
        s = lax.axis_index("s")
        w = s * NC + c
        pltpu.sync_copy(zeros_hbm, acc.at[pl.ds(s * N_STRIPE, N_STRIPE)])
        pltpu.sync_copy(ones_hbm, ones_v)
        pltpu.sync_copy(dst_hbm.at[pl.ds(w * kpt, kpt)], idx_v)
        plsc.subcore_barrier()
        for k in range(kpt):
            pltpu.sync_copy(ones_v, acc.at[idx_v.at[k]], add=True)
        plsc.subcore_barrier()
        pltpu.sync_copy(acc.at[pl.ds(s * N_STRIPE, N_STRIPE)],
                        out_hbm.at[c, pl.ds(s * N_STRIPE, N_STRIPE)])

    return deg_kernel(dst2, ones128, zeros128)


def _segment_sum(hs, src2, dst2, zeros128, split_edges):
    S = hs.shape[0]
    n_out = NC if split_edges else S
    spc = 1 if split_edges else S // NC
    kpt = src2.shape[0] // (NC * NS) if split_edges else src2.shape[0] // NS
    ngrp = kpt // SCH

    @functools.partial(
        pl.kernel,
        out_type=jax.ShapeDtypeStruct((n_out, N_PAD, LANES), jnp.float32),
        mesh=_sc_mesh(),
        scratch_types=[
            pltpu.VMEM((SCH, BE), jnp.int32),
            pltpu.VMEM((SCH, BE), jnp.int32),
            [pltpu.VMEM((BE, LANES), jnp.float32) for _ in range(NB)],
            pltpu.VMEM_SHARED((N_PAD, LANES), jnp.float32),
            [pltpu.SemaphoreType.DMA for _ in range(NB)],
            [pltpu.SemaphoreType.DMA for _ in range(NB)],
        ],
    )
    def seg_kernel(hs_hbm, src_hbm, dst_hbm, zeros_hbm, out_hbm,
                   sidx_v, didx_v, rows, acc, gsem, ssem):
        c = lax.axis_index("c")
        s = lax.axis_index("s")
        if split_edges:
            base = (s * NC + c) * kpt
        else:
            base = s * kpt

        for j in range(spc):
            if split_edges:
                sl = 0
                out_slot = c
            else:
                sl = c + NC * j
                out_slot = sl
            pltpu.sync_copy(zeros_hbm, acc.at[pl.ds(s * N_STRIPE, N_STRIPE)])
            plsc.subcore_barrier()

            def chunk_body(ch, _):
                row0 = pl.multiple_of(base + ch * SCH, SCH)
                pltpu.sync_copy(src_hbm.at[pl.ds(row0, SCH)], sidx_v)
                pltpu.sync_copy(dst_hbm.at[pl.ds(row0, SCH)], didx_v)
                gd = [None] * SCH
                sd = [None] * SCH

                def scat(b):
                    gd[b].wait()
                    sd[b] = pltpu.async_copy(
                        rows[b % NB], acc.at[didx_v.at[b]], ssem[b % NB],
                        add=True)

                for b in range(SCH):
                    if b >= NB:
                        sd[b - NB].wait()
                    gd[b] = pltpu.async_copy(
                        hs_hbm.at[sl].at[sidx_v.at[b]], rows[b % NB],
                        gsem[b % NB])
                    if b >= LAG:
                        scat(b - LAG)
                for b in range(SCH - LAG, SCH):
                    scat(b)
                for b in range(SCH - NB, SCH):
                    sd[b].wait()
                return _

            lax.fori_loop(0, ngrp, chunk_body, 0, unroll=False)
            plsc.subcore_barrier()
            pltpu.sync_copy(acc.at[pl.ds(s * N_STRIPE, N_STRIPE)],
                            out_hbm.at[out_slot, pl.ds(s * N_STRIPE, N_STRIPE)])
            plsc.subcore_barrier()

    return seg_kernel(hs, src2, dst2, zeros128)


_RB = 2000


def _dinv_from_deg(deg2):
    rb = 1280

    def body(deg_ref, o_ref):
        d = deg_ref[0, :, 0:1] + deg_ref[1, :, 0:1] + 1.0
        o_ref[...] = jnp.broadcast_to(lax.rsqrt(d), (rb, LANES))

    return pl.pallas_call(
        body,
        grid=(N_PAD // rb,),
        in_specs=[pl.BlockSpec((2, rb, LANES), lambda i: (0, i, 0))],
        out_specs=pl.BlockSpec((rb, LANES), lambda i: (i, 0)),
        out_shape=jax.ShapeDtypeStruct((N_PAD, LANES), jnp.float32),
    )(deg2)


def _stage_a(xin, W, dinv=None, gate=None):
    cin = xin.shape[1]
    S = W.shape[0] // LANES
    n_in = 2 + (dinv is not None) + (gate is not None)

    def body(*refs):
        x_ref, w_ref = refs[0], refs[1]
        o_ref = refs[-1]
        k = 2
        x = x_ref[...]
        if gate is not None:
            x = x * refs[k][...]
            k += 1
        h = lax.dot_general(x, w_ref[...], (((1,), (1,)), ((), ())),
                            preferred_element_type=jnp.float32)
        if dinv is not None:
            h = h * refs[k][...]
        o_ref[0] = h

    in_specs = [
        pl.BlockSpec((_RB, cin), lambda i, j: (i, 0)),
        pl.BlockSpec((LANES, cin), lambda i, j: (j, 0)),
    ]
    args = [xin, W]
    if gate is not None:
        in_specs.append(pl.BlockSpec((1, cin), lambda i, j: (0, 0)))
        args.append(gate)
    if dinv is not None:
        in_specs.append(pl.BlockSpec((_RB, LANES), lambda i, j: (i, 0)))
        args.append(dinv)

    return pl.pallas_call(
        body,
        grid=(N // _RB, S),
        in_specs=in_specs,
        out_specs=pl.BlockSpec((1, _RB, LANES), lambda i, j: (j, i, 0)),
        out_shape=jax.ShapeDtypeStruct((S, N, LANES), jnp.float32),
    )(*args)


def _scale_hs(h, dinv):
    S = h.shape[0]

    def body(h_ref, d_ref, o_ref):
        o_ref[0] = h_ref[0] * d_ref[...]

    return pl.pallas_call(
        body,
        grid=(N // _RB, S),
        in_specs=[pl.BlockSpec((1, _RB, LANES), lambda i, j: (j, i, 0)),
                  pl.BlockSpec((_RB, LANES), lambda i, j: (i, 0))],
        out_specs=pl.BlockSpec((1, _RB, LANES), lambda i, j: (j, i, 0)),
        out_shape=jax.ShapeDtypeStruct(h.shape, jnp.float32),
    )(h, dinv)


def _epilogue(agg, hs, dinv, alpha, beta, xin, Wr, skip, skip_gate,
              xin_gate, Wnext, split_edges):
    S = hs.shape[0]
    cout = S * LANES
    cin = xin.shape[1]
    a_blk = agg.shape[0]

    def body(*refs):
        k = 0

        def nxt():
            nonlocal k
            k += 1
            return refs[k - 1]

        agg_ref = nxt()
        hs_ref = nxt()
        d_ref = nxt()
        al_ref = nxt()
        be_ref = nxt()
        x_ref = nxt()
        wr_ref = nxt() if Wr is not None else None
        xg_ref = nxt() if xin_gate is not None else None
        sk_ref = nxt() if skip is not None else None
        sg_ref = nxt() if skip_gate is not None else None
        wn_ref = nxt() if Wnext is not None else None
        o_ref = nxt()
        on_ref = nxt() if Wnext is not None else None

        d = d_ref[...]
        al = al_ref[...]
        be = be_ref[...]
        cols = []
        for t in range(S):
            a = agg_ref[t] if a_blk == S else agg_ref[0] + agg_ref[1]
            lo = t * LANES
            yt = (a + hs_ref[t]) * d * al[:, lo:lo + LANES] + be[:, lo:lo + LANES]
            cols.append(jnp.maximum(yt, 0.0))
        y = cols[0] if S == 1 else jnp.concatenate(cols, axis=1)
        if Wr is None:
            res = x_ref[...]
            if xg_ref is not None:
                res = res * xg_ref[...]
        else:
            x = x_ref[...]
            if xg_ref is not None:
                x = x * xg_ref[...]
            res = lax.dot_general(x, wr_ref[...], (((1,), (1,)), ((), ())),
                                  preferred_element_type=jnp.float32)
        y = y + res
        if sk_ref is not None:
            sk = sk_ref[...]
            if sg_ref is not None:
                sk = sk * sg_ref[...]
            y = y + sk
        o_ref[...] = y
        if on_ref is not None:
            hn = lax.dot_general(y, wn_ref[...], (((1,), (1,)), ((), ())),
                                 preferred_element_type=jnp.float32)
            Sn = on_ref.shape[0]
            for t in range(Sn):
                on_ref[t] = hn[:, t * LANES:(t + 1) * LANES] * d

    in_specs = [
        pl.BlockSpec((a_blk, _RB, LANES), lambda i: (0, i, 0)),
        pl.BlockSpec((S, _RB, LANES), lambda i: (0, i, 0)),
        pl.BlockSpec((_RB, LANES), lambda i: (i, 0)),
        pl.BlockSpec((1, cout), lambda i: (0, 0)),
        pl.BlockSpec((1, cout), lambda i: (0, 0)),
    ]
    args = [agg, hs, dinv, alpha, beta]
    if Wr is None:
        in_specs.append(pl.BlockSpec((_RB, cout), lambda i: (i, 0)))
        args.append(xin)
    else:
        in_specs.append(pl.BlockSpec((_RB, cin), lambda i: (i, 0)))
        in_specs.append(pl.BlockSpec((cout, cin), lambda i: (0, 0)))
        args.extend([xin, Wr])
    if xin_gate is not None:
        in_specs.append(pl.BlockSpec((1, cin), lambda i: (0, 0)))
        args.append(xin_gate)
    if skip is not None:
        in_specs.append(pl.BlockSpec((_RB, cout), lambda i: (i, 0)))
        args.append(skip)
    if skip_gate is not None:
        in_specs.append(pl.BlockSpec((1, cout), lambda i: (0, 0)))
        args.append(skip_gate)

    out_specs = [pl.BlockSpec((_RB, cout), lambda i: (i, 0))]
    out_shape = [jax.ShapeDtypeStruct((N, cout), jnp.float32)]
    if Wnext is not None:
        in_specs.append(pl.BlockSpec((Wnext.shape[0], cout), lambda i: (0, 0)))
        args.append(Wnext)
        Sn = Wnext.shape[0] // LANES
        out_specs.append(pl.BlockSpec((Sn, _RB, LANES), lambda i: (0, i, 0)))
        out_shape.append(jax.ShapeDtypeStruct((Sn, N, LANES), jnp.float32))

    out = pl.pallas_call(
        body,
        grid=(N // _RB,),
        in_specs=in_specs,
        out_specs=out_specs,
        out_shape=out_shape,
    )(*args)
    return out if Wnext is not None else (out[0], None)


def _se_gate(h1, Wse1, bse1, Wse2, bse2):
    C = h1.shape[1]

    def pool_body(h_ref, o_ref):
        @pl.when(pl.program_id(0) == 0)
        def _():
            o_ref[...] = jnp.zeros_like(o_ref)
        o_ref[...] += jnp.sum(h_ref[...], axis=0, keepdims=True)

    pooled = pl.pallas_call(
        pool_body,
        grid=(N // _RB,),
        in_specs=[pl.BlockSpec((_RB, C), lambda i: (i, 0))],
        out_specs=pl.BlockSpec((1, C), lambda i: (0, 0)),
        out_shape=jax.ShapeDtypeStruct((1, C), jnp.float32),
    )(h1)

    def gate_body(p_ref, w1_ref, b1_ref, w2_ref, b2_ref, o_ref):
        p = p_ref[...] * (1.0 / N)
        t = lax.dot_general(p, w1_ref[...], (((1,), (1,)), ((), ())),
                            preferred_element_type=jnp.float32)
        t = jnp.maximum(t + b1_ref[...], 0.0)
        g = lax.dot_general(t, w2_ref[...], (((1,), (1,)), ((), ())),
                            preferred_element_type=jnp.float32)
        o_ref[...] = jax.nn.sigmoid(g + b2_ref[...])

    hid = Wse1.shape[0]
    return pl.pallas_call(
        gate_body,
        out_shape=jax.ShapeDtypeStruct((1, C), jnp.float32),
    )(pooled, Wse1, bse1.reshape(1, hid), Wse2, bse2.reshape(1, C))


def kernel(x, edge_index,
           Wc1, bc1, g1, be1, Wr1,
           Wc2, bc2, g2, be2, Wr2,
           Wc3, bc3, g3, be3,
           Wc4, bc4, g4, be4,
           Wc5, bc5, g5, be5, Wr5,
           Wc6, bc6, g6, be6, Wr6,
           Wse1, bse1, Wse2, bse2):
    src = edge_index[0]
    dst = edge_index[1]
    E = src.shape[0]
    align = NC * NS * BE * SCH
    epad = -(-E // align) * align
    pad = epad - E
    srcp = jnp.concatenate([src, jnp.zeros((pad,), src.dtype)])
    dstp = jnp.concatenate([dst, jnp.full((pad,), N, dst.dtype)])
    src2 = srcp.reshape(-1, BE)
    dst2 = dstp.reshape(-1, BE)

    ones128 = jnp.ones((BED, LANES), jnp.float32)
    zeros128 = jnp.zeros((N_STRIPE, LANES), jnp.float32)

    inv_bn = 1.0 / jnp.sqrt(1.0 + EPS)

    def consts(bc, g, be):
        alpha = (g * inv_bn).reshape(1, -1)
        beta = (bc * g * inv_bn + be).reshape(1, -1)
        return alpha, beta

    h1raw = _stage_a(x, Wc1)
    deg2 = _deg_count(dstp.reshape(-1, BED), ones128, zeros128)
    dinv = _dinv_from_deg(deg2)
    hs1 = _scale_hs(h1raw, dinv)

    def seg(hs):
        return _segment_sum(hs, src2, dst2, zeros128,
                            split_edges=(hs.shape[0] == 1))

    a1, b1 = consts(bc1, g1, be1)
    h1, _ = _epilogue(seg(hs1), hs1, dinv, a1, b1, x, Wr1,
                      None, None, None, None, False)
    gate = _se_gate(h1, Wse1, bse1, Wse2, bse2)

    hs2 = _stage_a(h1, Wc2, dinv, gate=gate)
    a2, b2 = consts(bc2, g2, be2)
    h2, hs3 = _epilogue(seg(hs2), hs2, dinv, a2, b2, h1, Wr2,
                        None, None, gate, Wc3, False)
    a3, b3 = consts(bc3, g3, be3)
    h3, hs4 = _epilogue(seg(hs3), hs3, dinv, a3, b3, h2, None,
                        None, None, None, Wc4, False)
    a4, b4 = consts(bc4, g4, be4)
    u2, hs5 = _epilogue(seg(hs4), hs4, dinv, a4, b4, h3, None,
                        h2, None, None, Wc5, False)
    a5, b5 = consts(bc5, g5, be5)
    u1, hs6 = _epilogue(seg(hs5), hs5, dinv, a5, b5, u2, Wr5,
                        h1, gate, None, Wc6, False)
    a6, b6 = consts(bc6, g6, be6)
    u0, _ = _epilogue(seg(hs6), hs6, dinv, a6, b6, u1, Wr6,
                      x, None, None, None, True)
    return u0


# speedup vs baseline: 1.0002x; 1.0002x over previous
import functools

import jax
import jax.numpy as jnp
from jax import lax
from jax.experimental import pallas as pl
from jax.experimental.pallas import tpu as pltpu
from jax.experimental.pallas import tpu_sc as plsc

N = 10000
EPS = 1e-5
LANES = 128
BE = 64
SCH = 32
NB = 4
LAG = 3
BED = 128
N_PAD = 10240
N_STRIPE = N_PAD // 16
NC, NS = 2, 16


def _sc_mesh():
    return plsc.VectorSubcoreMesh(
        core_axis_name="c", subcore_axis_name="s", num_cores=NC, num_subcores=NS)


def _deg_count(dst2, ones128, zeros128):
    kpt = dst2.shape[0] // (NC * NS)

    @functools.partial(
        pl.kernel,
        out_type=jax.ShapeDtypeStruct((NC, N_PAD, LANES), jnp.float32),
        mesh=_sc_mesh(),
        scratch_types=[
            pltpu.VMEM((kpt, BED), jnp.int32),
            pltpu.VMEM((BED, LANES), jnp.float32),
            pltpu.VMEM_SHARED((N_PAD, LANES), jnp.float32),
            [pltpu.SemaphoreType.DMA for _ in range(NB)],
        ],
    )
    def deg_kernel(dst_hbm, ones_hbm, zeros_hbm, out_hbm, idx_v, ones_v, acc, sem):
        c = lax.axis_index("c")
        s = lax.axis_index("s")
        w = s * NC + c
        pltpu.sync_copy(zeros_hbm, acc.at[pl.ds(s * N_STRIPE, N_STRIPE)])
        pltpu.sync_copy(ones_hbm, ones_v)
        pltpu.sync_copy(dst_hbm.at[pl.ds(w * kpt, kpt)], idx_v)
        plsc.subcore_barrier()
        sd = [None] * kpt
        for k in range(kpt):
            if k >= NB:
                sd[k - NB].wait()
            sd[k] = pltpu.async_copy(ones_v, acc.at[idx_v.at[k]],
                                     sem[k % NB], add=True)
        for k in range(kpt - NB, kpt):
            sd[k].wait()
        plsc.subcore_barrier()
        pltpu.sync_copy(acc.at[pl.ds(s * N_STRIPE, N_STRIPE)],
                        out_hbm.at[c, pl.ds(s * N_STRIPE, N_STRIPE)])

    return deg_kernel(dst2, ones128, zeros128)


def _segment_sum(hs, src2, dst2, zeros128, split_edges):
    S = hs.shape[0]
    n_out = NC if split_edges else S
    spc = 1 if split_edges else S // NC
    kpt = src2.shape[0] // (NC * NS) if split_edges else src2.shape[0] // NS
    ngrp = kpt // SCH

    @functools.partial(
        pl.kernel,
        out_type=jax.ShapeDtypeStruct((n_out, N_PAD, LANES), jnp.float32),
        mesh=_sc_mesh(),
        scratch_types=[
            pltpu.VMEM((SCH, BE), jnp.int32),
            pltpu.VMEM((SCH, BE), jnp.int32),
            [pltpu.VMEM((BE, LANES), jnp.float32) for _ in range(NB)],
            pltpu.VMEM_SHARED((N_PAD, LANES), jnp.float32),
            [pltpu.SemaphoreType.DMA for _ in range(NB)],
            [pltpu.SemaphoreType.DMA for _ in range(NB)],
        ],
    )
    def seg_kernel(hs_hbm, src_hbm, dst_hbm, zeros_hbm, out_hbm,
                   sidx_v, didx_v, rows, acc, gsem, ssem):
        c = lax.axis_index("c")
        s = lax.axis_index("s")
        if split_edges:
            base = (s * NC + c) * kpt
        else:
            base = s * kpt

        for j in range(spc):
            if split_edges:
                sl = 0
                out_slot = c
            else:
                sl = c + NC * j
                out_slot = sl
            pltpu.sync_copy(zeros_hbm, acc.at[pl.ds(s * N_STRIPE, N_STRIPE)])
            plsc.subcore_barrier()

            def chunk_body(ch, _):
                row0 = pl.multiple_of(base + ch * SCH, SCH)
                pltpu.sync_copy(src_hbm.at[pl.ds(row0, SCH)], sidx_v)
                pltpu.sync_copy(dst_hbm.at[pl.ds(row0, SCH)], didx_v)
                gd = [None] * SCH
                sd = [None] * SCH

                def scat(b):
                    gd[b].wait()
                    sd[b] = pltpu.async_copy(
                        rows[b % NB], acc.at[didx_v.at[b]], ssem[b % NB],
                        add=True)

                for b in range(SCH):
                    if b >= NB:
                        sd[b - NB].wait()
                    gd[b] = pltpu.async_copy(
                        hs_hbm.at[sl].at[sidx_v.at[b]], rows[b % NB],
                        gsem[b % NB])
                    if b >= LAG:
                        scat(b - LAG)
                for b in range(SCH - LAG, SCH):
                    scat(b)
                for b in range(SCH - NB, SCH):
                    sd[b].wait()
                return _

            lax.fori_loop(0, ngrp, chunk_body, 0, unroll=False)
            plsc.subcore_barrier()
            pltpu.sync_copy(acc.at[pl.ds(s * N_STRIPE, N_STRIPE)],
                            out_hbm.at[out_slot, pl.ds(s * N_STRIPE, N_STRIPE)])
            plsc.subcore_barrier()

    return seg_kernel(hs, src2, dst2, zeros128)


_RB = 2000


def _dinv_from_deg(deg2):
    rb = 1280

    def body(deg_ref, o_ref):
        d = deg_ref[0, :, 0:1] + deg_ref[1, :, 0:1] + 1.0
        o_ref[...] = jnp.broadcast_to(lax.rsqrt(d), (rb, LANES))

    return pl.pallas_call(
        body,
        grid=(N_PAD // rb,),
        in_specs=[pl.BlockSpec((2, rb, LANES), lambda i: (0, i, 0))],
        out_specs=pl.BlockSpec((rb, LANES), lambda i: (i, 0)),
        out_shape=jax.ShapeDtypeStruct((N_PAD, LANES), jnp.float32),
    )(deg2)


def _stage_a(xin, W, dinv=None, gate=None):
    cin = xin.shape[1]
    S = W.shape[0] // LANES
    n_in = 2 + (dinv is not None) + (gate is not None)

    def body(*refs):
        x_ref, w_ref = refs[0], refs[1]
        o_ref = refs[-1]
        k = 2
        x = x_ref[...]
        if gate is not None:
            x = x * refs[k][...]
            k += 1
        h = lax.dot_general(x, w_ref[...], (((1,), (1,)), ((), ())),
                            preferred_element_type=jnp.float32)
        if dinv is not None:
            h = h * refs[k][...]
        o_ref[0] = h

    in_specs = [
        pl.BlockSpec((_RB, cin), lambda i, j: (i, 0)),
        pl.BlockSpec((LANES, cin), lambda i, j: (j, 0)),
    ]
    args = [xin, W]
    if gate is not None:
        in_specs.append(pl.BlockSpec((1, cin), lambda i, j: (0, 0)))
        args.append(gate)
    if dinv is not None:
        in_specs.append(pl.BlockSpec((_RB, LANES), lambda i, j: (i, 0)))
        args.append(dinv)

    return pl.pallas_call(
        body,
        grid=(N // _RB, S),
        in_specs=in_specs,
        out_specs=pl.BlockSpec((1, _RB, LANES), lambda i, j: (j, i, 0)),
        out_shape=jax.ShapeDtypeStruct((S, N, LANES), jnp.float32),
    )(*args)


def _scale_hs(h, dinv):
    S = h.shape[0]

    def body(h_ref, d_ref, o_ref):
        o_ref[0] = h_ref[0] * d_ref[...]

    return pl.pallas_call(
        body,
        grid=(N // _RB, S),
        in_specs=[pl.BlockSpec((1, _RB, LANES), lambda i, j: (j, i, 0)),
                  pl.BlockSpec((_RB, LANES), lambda i, j: (i, 0))],
        out_specs=pl.BlockSpec((1, _RB, LANES), lambda i, j: (j, i, 0)),
        out_shape=jax.ShapeDtypeStruct(h.shape, jnp.float32),
    )(h, dinv)


def _epilogue(agg, hs, dinv, alpha, beta, xin, Wr, skip, skip_gate,
              xin_gate, Wnext, split_edges):
    S = hs.shape[0]
    cout = S * LANES
    cin = xin.shape[1]
    a_blk = agg.shape[0]

    def body(*refs):
        k = 0

        def nxt():
            nonlocal k
            k += 1
            return refs[k - 1]

        agg_ref = nxt()
        hs_ref = nxt()
        d_ref = nxt()
        al_ref = nxt()
        be_ref = nxt()
        x_ref = nxt()
        wr_ref = nxt() if Wr is not None else None
        xg_ref = nxt() if xin_gate is not None else None
        sk_ref = nxt() if skip is not None else None
        sg_ref = nxt() if skip_gate is not None else None
        wn_ref = nxt() if Wnext is not None else None
        o_ref = nxt()
        on_ref = nxt() if Wnext is not None else None

        d = d_ref[...]
        al = al_ref[...]
        be = be_ref[...]
        cols = []
        for t in range(S):
            a = agg_ref[t] if a_blk == S else agg_ref[0] + agg_ref[1]
            lo = t * LANES
            yt = (a + hs_ref[t]) * d * al[:, lo:lo + LANES] + be[:, lo:lo + LANES]
            cols.append(jnp.maximum(yt, 0.0))
        y = cols[0] if S == 1 else jnp.concatenate(cols, axis=1)
        if Wr is None:
            res = x_ref[...]
            if xg_ref is not None:
                res = res * xg_ref[...]
        else:
            x = x_ref[...]
            if xg_ref is not None:
                x = x * xg_ref[...]
            res = lax.dot_general(x, wr_ref[...], (((1,), (1,)), ((), ())),
                                  preferred_element_type=jnp.float32)
        y = y + res
        if sk_ref is not None:
            sk = sk_ref[...]
            if sg_ref is not None:
                sk = sk * sg_ref[...]
            y = y + sk
        o_ref[...] = y
        if on_ref is not None:
            hn = lax.dot_general(y, wn_ref[...], (((1,), (1,)), ((), ())),
                                 preferred_element_type=jnp.float32)
            Sn = on_ref.shape[0]
            for t in range(Sn):
                on_ref[t] = hn[:, t * LANES:(t + 1) * LANES] * d

    in_specs = [
        pl.BlockSpec((a_blk, _RB, LANES), lambda i: (0, i, 0)),
        pl.BlockSpec((S, _RB, LANES), lambda i: (0, i, 0)),
        pl.BlockSpec((_RB, LANES), lambda i: (i, 0)),
        pl.BlockSpec((1, cout), lambda i: (0, 0)),
        pl.BlockSpec((1, cout), lambda i: (0, 0)),
    ]
    args = [agg, hs, dinv, alpha, beta]
    if Wr is None:
        in_specs.append(pl.BlockSpec((_RB, cout), lambda i: (i, 0)))
        args.append(xin)
    else:
        in_specs.append(pl.BlockSpec((_RB, cin), lambda i: (i, 0)))
        in_specs.append(pl.BlockSpec((cout, cin), lambda i: (0, 0)))
        args.extend([xin, Wr])
    if xin_gate is not None:
        in_specs.append(pl.BlockSpec((1, cin), lambda i: (0, 0)))
        args.append(xin_gate)
    if skip is not None:
        in_specs.append(pl.BlockSpec((_RB, cout), lambda i: (i, 0)))
        args.append(skip)
    if skip_gate is not None:
        in_specs.append(pl.BlockSpec((1, cout), lambda i: (0, 0)))
        args.append(skip_gate)

    out_specs = [pl.BlockSpec((_RB, cout), lambda i: (i, 0))]
    out_shape = [jax.ShapeDtypeStruct((N, cout), jnp.float32)]
    if Wnext is not None:
        in_specs.append(pl.BlockSpec((Wnext.shape[0], cout), lambda i: (0, 0)))
        args.append(Wnext)
        Sn = Wnext.shape[0] // LANES
        out_specs.append(pl.BlockSpec((Sn, _RB, LANES), lambda i: (0, i, 0)))
        out_shape.append(jax.ShapeDtypeStruct((Sn, N, LANES), jnp.float32))

    out = pl.pallas_call(
        body,
        grid=(N // _RB,),
        in_specs=in_specs,
        out_specs=out_specs,
        out_shape=out_shape,
    )(*args)
    return out if Wnext is not None else (out[0], None)


def _se_gate(h1, Wse1, bse1, Wse2, bse2):
    C = h1.shape[1]

    def pool_body(h_ref, o_ref):
        @pl.when(pl.program_id(0) == 0)
        def _():
            o_ref[...] = jnp.zeros_like(o_ref)
        o_ref[...] += jnp.sum(h_ref[...], axis=0, keepdims=True)

    pooled = pl.pallas_call(
        pool_body,
        grid=(N // _RB,),
        in_specs=[pl.BlockSpec((_RB, C), lambda i: (i, 0))],
        out_specs=pl.BlockSpec((1, C), lambda i: (0, 0)),
        out_shape=jax.ShapeDtypeStruct((1, C), jnp.float32),
    )(h1)

    def gate_body(p_ref, w1_ref, b1_ref, w2_ref, b2_ref, o_ref):
        p = p_ref[...] * (1.0 / N)
        t = lax.dot_general(p, w1_ref[...], (((1,), (1,)), ((), ())),
                            preferred_element_type=jnp.float32)
        t = jnp.maximum(t + b1_ref[...], 0.0)
        g = lax.dot_general(t, w2_ref[...], (((1,), (1,)), ((), ())),
                            preferred_element_type=jnp.float32)
        o_ref[...] = jax.nn.sigmoid(g + b2_ref[...])

    hid = Wse1.shape[0]
    return pl.pallas_call(
        gate_body,
        out_shape=jax.ShapeDtypeStruct((1, C), jnp.float32),
    )(pooled, Wse1, bse1.reshape(1, hid), Wse2, bse2.reshape(1, C))


def kernel(x, edge_index,
           Wc1, bc1, g1, be1, Wr1,
           Wc2, bc2, g2, be2, Wr2,
           Wc3, bc3, g3, be3,
           Wc4, bc4, g4, be4,
           Wc5, bc5, g5, be5, Wr5,
           Wc6, bc6, g6, be6, Wr6,
           Wse1, bse1, Wse2, bse2):
    src = edge_index[0]
    dst = edge_index[1]
    E = src.shape[0]
    align = NC * NS * BE * SCH
    epad = -(-E // align) * align
    pad = epad - E
    srcp = jnp.concatenate([src, jnp.zeros((pad,), src.dtype)])
    dstp = jnp.concatenate([dst, jnp.full((pad,), N, dst.dtype)])
    src2 = srcp.reshape(-1, BE)
    dst2 = dstp.reshape(-1, BE)

    ones128 = jnp.ones((BED, LANES), jnp.float32)
    zeros128 = jnp.zeros((N_STRIPE, LANES), jnp.float32)

    inv_bn = 1.0 / jnp.sqrt(1.0 + EPS)

    def consts(bc, g, be):
        alpha = (g * inv_bn).reshape(1, -1)
        beta = (bc * g * inv_bn + be).reshape(1, -1)
        return alpha, beta

    h1raw = _stage_a(x, Wc1)
    deg2 = _deg_count(dstp.reshape(-1, BED), ones128, zeros128)
    dinv = _dinv_from_deg(deg2)
    hs1 = _scale_hs(h1raw, dinv)

    def seg(hs):
        return _segment_sum(hs, src2, dst2, zeros128,
                            split_edges=(hs.shape[0] == 1))

    a1, b1 = consts(bc1, g1, be1)
    h1, _ = _epilogue(seg(hs1), hs1, dinv, a1, b1, x, Wr1,
                      None, None, None, None, False)
    gate = _se_gate(h1, Wse1, bse1, Wse2, bse2)

    hs2 = _stage_a(h1, Wc2, dinv, gate=gate)
    a2, b2 = consts(bc2, g2, be2)
    h2, hs3 = _epilogue(seg(hs2), hs2, dinv, a2, b2, h1, Wr2,
                        None, None, gate, Wc3, False)
    a3, b3 = consts(bc3, g3, be3)
    h3, hs4 = _epilogue(seg(hs3), hs3, dinv, a3, b3, h2, None,
                        None, None, None, Wc4, False)
    a4, b4 = consts(bc4, g4, be4)
    u2, hs5 = _epilogue(seg(hs4), hs4, dinv, a4, b4, h3, None,
                        h2, None, None, Wc5, False)
    a5, b5 = consts(bc5, g5, be5)
    u1, hs6 = _epilogue(seg(hs5), hs5, dinv, a5, b5, u2, Wr5,
                        h1, gate, None, Wc6, False)
    a6, b6 = consts(bc6, g6, be6)
    u0, _ = _epilogue(seg(hs6), hs6, dinv, a6, b6, u1, Wr6,
                      x, None, None, None, True)
    return u0
